# baseline (speedup 1.0000x reference)
import functools

import jax
import jax.numpy as jnp
from jax import lax
from jax.experimental import pallas as pl
from jax.experimental.pallas import tpu as pltpu
from jax.experimental.pallas import tpu_sc as plsc

_K = 5
_BQ = 256


def _simtopk_kernel(q_ref, k_ref, idx_ref):
    q = q_ref[...]
    k = k_ref[...]
    qn = q / jnp.maximum(jnp.sqrt(jnp.sum(q * q, axis=1, keepdims=True)), 1e-12)
    kn = k / jnp.maximum(jnp.sqrt(jnp.sum(k * k, axis=1, keepdims=True)), 1e-12)
    sim = jnp.dot(qn, kn.T, preferred_element_type=jnp.float32)
    cols = jax.lax.broadcasted_iota(jnp.int32, sim.shape, 1)
    picks = []
    for _ in range(_K):
        m = jnp.max(sim, axis=1, keepdims=True)
        a = jnp.min(jnp.where(sim == m, cols, jnp.int32(2**30)), axis=1)
        picks.append(a)
        sim = jnp.where(cols == a[:, None], -jnp.inf, sim)
    idx_ref[...] = jnp.stack(picks, axis=1)


try:
    _SC_INFO = plsc.get_sparse_core_info()
    _NC, _NS = _SC_INFO.num_cores, _SC_INFO.num_subcores
except Exception:
    _NC, _NS = 2, 16
_NW = _NC * _NS
_CHUNK = 8
_NBUF = 5


_QG = 8
_CPW = 4


def _sc_gather(table_hbm, idx_hbm, out_hbm, idx_v, rid_v,
               buf0, buf1, buf2, buf3, buf4,
               gsem0, gsem1, gsem2, gsem3, gsem4,
               wsem0, wsem1, wsem2, wsem3, wsem4):
    n_prompts = table_hbm.shape[0] // _K
    bufs = (buf0, buf1, buf2, buf3, buf4)
    gsems = (gsem0, gsem1, gsem2, gsem3, gsem4)
    wsems = (wsem0, wsem1, wsem2, wsem3, wsem4)
    b = idx_hbm.shape[0]
    b_per_w = b // _NW
    q_per_w = b_per_w // _K
    n_groups = q_per_w // _QG
    wid = lax.axis_index("s") * _NC + lax.axis_index("c")
    base = wid * b_per_w
    qb = wid * q_per_w
    pltpu.sync_copy(idx_hbm.at[pl.ds(base, b_per_w)], idx_v)
    iota = lax.iota(jnp.int32, 16)
    hi = lax.shift_right_logical(iota, 3)
    lo = lax.bitwise_and(iota, 7)

    def expand(g16):
        for j in range(_K):
            sel = plsc.load_gather(idx_v, [_K * (16 * g16 + iota) + j])
            for i in range(_K):
                pos = 400 * g16 + 200 * hi + (_K * j + i) * _QG + lo
                plsc.store_scatter(rid_v, [pos], n_prompts * i + sel)

    rows = _CPW * _QG
    n_chunks = n_groups * _K * _K // _CPW
    ghandles = [None] * n_chunks
    whandles = [[] for _ in range(n_chunks)]
    expand(0)
    for c in range(min(_NBUF, n_chunks)):
        ghandles[c] = pltpu.async_copy(
            table_hbm.at[rid_v.at[pl.ds(c * rows, rows)]],
            bufs[c % _NBUF], gsems[c % _NBUF])
    for g16 in range(1, q_per_w // 16):
        expand(g16)
    for c in range(n_chunks):
        p = c % _NBUF
        ghandles[c].wait()
        for m in range(_CPW):
            g, co = divmod(c * _CPW + m, _K * _K)
            j, i = divmod(co, _K)
            whandles[c].append(pltpu.async_copy(
                bufs[p].at[pl.ds(m * _QG, _QG)],
                out_hbm.at[j, i, pl.ds(qb + g * _QG, _QG)], wsems[p]))
        nxt = c + _NBUF
        if nxt < n_chunks:
            for h in whandles[nxt - _NBUF]:
                h.wait()
            whandles[nxt - _NBUF] = []
            ghandles[nxt] = pltpu.async_copy(
                table_hbm.at[rid_v.at[pl.ds(nxt * rows, rows)]],
                bufs[nxt % _NBUF], gsems[nxt % _NBUF])
    for c in range(n_chunks):
        for h in whandles[c]:
            h.wait()


def kernel(query, top_k, prompts, prompt_keys):
    del top_k
    nq, d = query.shape
    n, k, _ = prompts.shape

    indices = pl.pallas_call(
        _simtopk_kernel,
        grid=(nq // _BQ,),
        in_specs=[
            pl.BlockSpec((_BQ, d), lambda i: (i, 0)),
            pl.BlockSpec((n, d), lambda i: (0, 0)),
        ],
        out_specs=pl.BlockSpec((_BQ, _K), lambda i: (i, 0)),
        out_shape=jax.ShapeDtypeStruct((nq, _K), jnp.int32),
    )(query, prompt_keys)

    b = nq * _K
    gather = functools.partial(
        pl.kernel,
        out_type=jax.ShapeDtypeStruct((_K, k, nq, d), jnp.float32),
        mesh=plsc.VectorSubcoreMesh(core_axis_name="c", subcore_axis_name="s"),
        compiler_params=pltpu.CompilerParams(
            needs_layout_passes=False, use_tc_tiling_on_sc=True),
        scratch_types=(
            [
                pltpu.VMEM((b // _NW,), jnp.int32),
                pltpu.VMEM((b // _NW * _K,), jnp.int32),
            ]
            + [pltpu.VMEM((_CPW * _QG, d), jnp.float32)] * _NBUF
            + [pltpu.SemaphoreType.DMA] * (2 * _NBUF)
        ),
    )(_sc_gather)
    table = jnp.transpose(prompts, (1, 0, 2)).reshape(k * n, d)
    gathered = gather(table, indices.reshape(b))

    return jnp.transpose(gathered, (2, 0, 1, 3)), indices


# speedup vs baseline: 1.0109x; 1.0109x over previous
import functools

import jax
import jax.numpy as jnp
from jax import lax
from jax.experimental import pallas as pl
from jax.experimental.pallas import tpu as pltpu
from jax.experimental.pallas import tpu_sc as plsc

_K = 5
_BQ = 256


def _simtopk_kernel(q_ref, k_ref, idx_ref):
    q = q_ref[...]
    k = k_ref[...]
    qn = q / jnp.maximum(jnp.sqrt(jnp.sum(q * q, axis=1, keepdims=True)), 1e-12)
    kn = k / jnp.maximum(jnp.sqrt(jnp.sum(k * k, axis=1, keepdims=True)), 1e-12)
    sim = jnp.dot(qn, kn.T, preferred_element_type=jnp.float32)
    cols = jax.lax.broadcasted_iota(jnp.int32, sim.shape, 1)
    picks = []
    for _ in range(_K):
        m = jnp.max(sim, axis=1, keepdims=True)
        a = jnp.min(jnp.where(sim == m, cols, jnp.int32(2**30)), axis=1)
        picks.append(a)
        sim = jnp.where(cols == a[:, None], -jnp.inf, sim)
    idx_ref[...] = jnp.stack(picks, axis=1)


try:
    _SC_INFO = plsc.get_sparse_core_info()
    _NC, _NS = _SC_INFO.num_cores, _SC_INFO.num_subcores
except Exception:
    _NC, _NS = 2, 16
_NW = _NC * _NS
_CHUNK = 8
_NBUF = 5


_QG = 16
_CPW = 2


def _sc_gather(table_hbm, idx_hbm, out_hbm, idx_v, rid_v,
               buf0, buf1, buf2, buf3, buf4,
               gsem0, gsem1, gsem2, gsem3, gsem4,
               wsem0, wsem1, wsem2, wsem3, wsem4):
    n_prompts = table_hbm.shape[0] // _K
    bufs = (buf0, buf1, buf2, buf3, buf4)
    gsems = (gsem0, gsem1, gsem2, gsem3, gsem4)
    wsems = (wsem0, wsem1, wsem2, wsem3, wsem4)
    b = idx_hbm.shape[0]
    b_per_w = b // _NW
    q_per_w = b_per_w // _K
    n_groups = q_per_w // _QG
    wid = lax.axis_index("s") * _NC + lax.axis_index("c")
    base = wid * b_per_w
    qb = wid * q_per_w
    pltpu.sync_copy(idx_hbm.at[pl.ds(base, b_per_w)], idx_v)
    iota = lax.iota(jnp.int32, 16)

    def expand(g16):
        q16 = 16 * g16 + iota
        g_vec = lax.shift_right_logical(q16, _QG.bit_length() - 1)
        qq = lax.bitwise_and(q16, _QG - 1)
        for j in range(_K):
            sel = plsc.load_gather(idx_v, [_K * q16 + j])
            for i in range(_K):
                pos = (g_vec * (_K * _K) + (_K * j + i)) * _QG + qq
                plsc.store_scatter(rid_v, [pos], n_prompts * i + sel)

    rows = _CPW * _QG
    n_chunks = n_groups * _K * _K // _CPW
    ghandles = [None] * n_chunks
    whandles = [[] for _ in range(n_chunks)]
    expand(0)
    for c in range(min(_NBUF, n_chunks)):
        ghandles[c] = pltpu.async_copy(
            table_hbm.at[rid_v.at[pl.ds(c * rows, rows)]],
            bufs[c % _NBUF], gsems[c % _NBUF])
    for g16 in range(1, q_per_w // 16):
        expand(g16)
    for c in range(n_chunks):
        p = c % _NBUF
        ghandles[c].wait()
        for m in range(_CPW):
            g, co = divmod(c * _CPW + m, _K * _K)
            j, i = divmod(co, _K)
            whandles[c].append(pltpu.async_copy(
                bufs[p].at[pl.ds(m * _QG, _QG)],
                out_hbm.at[j, i, pl.ds(qb + g * _QG, _QG)], wsems[p]))
        nxt = c + _NBUF
        if nxt < n_chunks:
            for h in whandles[nxt - _NBUF]:
                h.wait()
            whandles[nxt - _NBUF] = []
            ghandles[nxt] = pltpu.async_copy(
                table_hbm.at[rid_v.at[pl.ds(nxt * rows, rows)]],
                bufs[nxt % _NBUF], gsems[nxt % _NBUF])
    for c in range(n_chunks):
        for h in whandles[c]:
            h.wait()


def kernel(query, top_k, prompts, prompt_keys):
    del top_k
    nq, d = query.shape
    n, k, _ = prompts.shape

    indices = pl.pallas_call(
        _simtopk_kernel,
        grid=(nq // _BQ,),
        in_specs=[
            pl.BlockSpec((_BQ, d), lambda i: (i, 0)),
            pl.BlockSpec((n, d), lambda i: (0, 0)),
        ],
        out_specs=pl.BlockSpec((_BQ, _K), lambda i: (i, 0)),
        out_shape=jax.ShapeDtypeStruct((nq, _K), jnp.int32),
    )(query, prompt_keys)

    b = nq * _K
    gather = functools.partial(
        pl.kernel,
        out_type=jax.ShapeDtypeStruct((_K, k, nq, d), jnp.float32),
        mesh=plsc.VectorSubcoreMesh(core_axis_name="c", subcore_axis_name="s"),
        compiler_params=pltpu.CompilerParams(
            needs_layout_passes=False, use_tc_tiling_on_sc=True),
        scratch_types=(
            [
                pltpu.VMEM((b // _NW,), jnp.int32),
                pltpu.VMEM((b // _NW * _K,), jnp.int32),
            ]
            + [pltpu.VMEM((_CPW * _QG, d), jnp.float32)] * _NBUF
            + [pltpu.SemaphoreType.DMA] * (2 * _NBUF)
        ),
    )(_sc_gather)
    table = jnp.transpose(prompts, (1, 0, 2)).reshape(k * n, d)
    gathered = gather(table, indices.reshape(b))

    return jnp.transpose(gathered, (2, 0, 1, 3)), indices


# speedup vs baseline: 1.0372x; 1.0260x over previous
import functools

import jax
import jax.numpy as jnp
from jax import lax
from jax.experimental import pallas as pl
from jax.experimental.pallas import tpu as pltpu
from jax.experimental.pallas import tpu_sc as plsc

_K = 5
_BQ = 256


def _simtopk_kernel(q_ref, k_ref, idx_ref):
    q = q_ref[...]
    k = k_ref[...]
    qn = q / jnp.maximum(jnp.sqrt(jnp.sum(q * q, axis=1, keepdims=True)), 1e-12)
    kn = k / jnp.maximum(jnp.sqrt(jnp.sum(k * k, axis=1, keepdims=True)), 1e-12)
    sim = jnp.dot(qn, kn.T, preferred_element_type=jnp.float32)
    cols = jax.lax.broadcasted_iota(jnp.int32, sim.shape, 1)
    picks = []
    for _ in range(_K):
        a = jnp.argmax(sim, axis=1).astype(jnp.int32)
        picks.append(a)
        sim = jnp.where(cols == a[:, None], -jnp.inf, sim)
    idx_ref[...] = jnp.stack(picks, axis=1)


try:
    _SC_INFO = plsc.get_sparse_core_info()
    _NC, _NS = _SC_INFO.num_cores, _SC_INFO.num_subcores
except Exception:
    _NC, _NS = 2, 16
_NW = _NC * _NS
_CHUNK = 8
_NBUF = 5


_QG = 16
_CPW = 2


def _sc_gather(table_hbm, idx_hbm, out_hbm, idx_v, rid_v,
               buf0, buf1, buf2, buf3, buf4,
               gsem0, gsem1, gsem2, gsem3, gsem4,
               wsem0, wsem1, wsem2, wsem3, wsem4):
    n_prompts = table_hbm.shape[0] // _K
    bufs = (buf0, buf1, buf2, buf3, buf4)
    gsems = (gsem0, gsem1, gsem2, gsem3, gsem4)
    wsems = (wsem0, wsem1, wsem2, wsem3, wsem4)
    b = idx_hbm.shape[0]
    b_per_w = b // _NW
    q_per_w = b_per_w // _K
    n_groups = q_per_w // _QG
    wid = lax.axis_index("s") * _NC + lax.axis_index("c")
    base = wid * b_per_w
    qb = wid * q_per_w
    pltpu.sync_copy(idx_hbm.at[pl.ds(base, b_per_w)], idx_v)
    iota = lax.iota(jnp.int32, 16)

    def expand(g16):
        q16 = 16 * g16 + iota
        g_vec = lax.shift_right_logical(q16, _QG.bit_length() - 1)
        qq = lax.bitwise_and(q16, _QG - 1)
        for j in range(_K):
            sel = plsc.load_gather(idx_v, [_K * q16 + j])
            for i in range(_K):
                pos = (g_vec * (_K * _K) + (_K * j + i)) * _QG + qq
                plsc.store_scatter(rid_v, [pos], n_prompts * i + sel)

    rows = _CPW * _QG
    n_chunks = n_groups * _K * _K // _CPW
    ghandles = [None] * n_chunks
    whandles = [[] for _ in range(n_chunks)]
    expand(0)
    for c in range(min(_NBUF, n_chunks)):
        ghandles[c] = pltpu.async_copy(
            table_hbm.at[rid_v.at[pl.ds(c * rows, rows)]],
            bufs[c % _NBUF], gsems[c % _NBUF])
    for g16 in range(1, q_per_w // 16):
        expand(g16)
    for c in range(n_chunks):
        p = c % _NBUF
        ghandles[c].wait()
        for m in range(_CPW):
            g, co = divmod(c * _CPW + m, _K * _K)
            j, i = divmod(co, _K)
            whandles[c].append(pltpu.async_copy(
                bufs[p].at[pl.ds(m * _QG, _QG)],
                out_hbm.at[j, i, pl.ds(qb + g * _QG, _QG)], wsems[p]))
        nxt = c + _NBUF
        if nxt < n_chunks:
            for h in whandles[nxt - _NBUF]:
                h.wait()
            whandles[nxt - _NBUF] = []
            ghandles[nxt] = pltpu.async_copy(
                table_hbm.at[rid_v.at[pl.ds(nxt * rows, rows)]],
                bufs[nxt % _NBUF], gsems[nxt % _NBUF])
    for c in range(n_chunks):
        for h in whandles[c]:
            h.wait()


def kernel(query, top_k, prompts, prompt_keys):
    del top_k
    nq, d = query.shape
    n, k, _ = prompts.shape

    indices = pl.pallas_call(
        _simtopk_kernel,
        grid=(nq // _BQ,),
        in_specs=[
            pl.BlockSpec((_BQ, d), lambda i: (i, 0)),
            pl.BlockSpec((n, d), lambda i: (0, 0)),
        ],
        out_specs=pl.BlockSpec((_BQ, _K), lambda i: (i, 0)),
        out_shape=jax.ShapeDtypeStruct((nq, _K), jnp.int32),
    )(query, prompt_keys)

    b = nq * _K
    gather = functools.partial(
        pl.kernel,
        out_type=jax.ShapeDtypeStruct((_K, k, nq, d), jnp.float32),
        mesh=plsc.VectorSubcoreMesh(core_axis_name="c", subcore_axis_name="s"),
        compiler_params=pltpu.CompilerParams(
            needs_layout_passes=False, use_tc_tiling_on_sc=True),
        scratch_types=(
            [
                pltpu.VMEM((b // _NW,), jnp.int32),
                pltpu.VMEM((b // _NW * _K,), jnp.int32),
            ]
            + [pltpu.VMEM((_CPW * _QG, d), jnp.float32)] * _NBUF
            + [pltpu.SemaphoreType.DMA] * (2 * _NBUF)
        ),
    )(_sc_gather)
    table = jnp.transpose(prompts, (1, 0, 2)).reshape(k * n, d)
    gathered = gather(table, indices.reshape(b))

    return jnp.transpose(gathered, (2, 0, 1, 3)), indices


# speedup vs baseline: 1.0409x; 1.0036x over previous
import functools

import jax
import jax.numpy as jnp
from jax import lax
from jax.experimental import pallas as pl
from jax.experimental.pallas import tpu as pltpu
from jax.experimental.pallas import tpu_sc as plsc

_K = 5
_BQ = 256


def _simtopk_kernel(q_ref, k_ref, idx_ref):
    q = q_ref[...]
    k = k_ref[...]
    qn = q / jnp.maximum(jnp.sqrt(jnp.sum(q * q, axis=1, keepdims=True)), 1e-12)
    kn = k / jnp.maximum(jnp.sqrt(jnp.sum(k * k, axis=1, keepdims=True)), 1e-12)
    sim = jnp.dot(qn, kn.T, preferred_element_type=jnp.float32)
    cols = jax.lax.broadcasted_iota(jnp.int32, sim.shape, 1)
    picks = []
    for _ in range(_K):
        a = jnp.argmax(sim, axis=1).astype(jnp.int32)
        picks.append(a)
        sim = jnp.where(cols == a[:, None], -jnp.inf, sim)
    idx_ref[...] = jnp.stack(picks, axis=1)


try:
    _SC_INFO = plsc.get_sparse_core_info()
    _NC, _NS = _SC_INFO.num_cores, _SC_INFO.num_subcores
except Exception:
    _NC, _NS = 2, 16
_NW = _NC * _NS
_NBUF = 5
_QG = 16
_CPW = 2


def _sc_gather(table_hbm, idx_hbm, out_hbm, idx_v, rid_v,
               buf0, buf1, buf2, buf3, buf4,
               gsem0, gsem1, gsem2, gsem3, gsem4,
               wsem0, wsem1, wsem2, wsem3, wsem4):
    n_prompts = table_hbm.shape[0] // _K
    bufs = (buf0, buf1, buf2, buf3, buf4)
    gsems = (gsem0, gsem1, gsem2, gsem3, gsem4)
    wsems = (wsem0, wsem1, wsem2, wsem3, wsem4)
    b = idx_hbm.shape[0]
    b_per_w = b // _NW
    q_per_w = b_per_w // _K
    n_groups = q_per_w // _QG
    wid = lax.axis_index("s") * _NC + lax.axis_index("c")
    base = wid * b_per_w
    qb = wid * q_per_w
    pltpu.sync_copy(idx_hbm.at[pl.ds(base, b_per_w)], idx_v)
    iota = lax.iota(jnp.int32, 16)

    def expand(g16):
        q16 = 16 * g16 + iota
        g_vec = lax.shift_right_logical(q16, _QG.bit_length() - 1)
        qq = lax.bitwise_and(q16, _QG - 1)
        for j in range(_K):
            sel = plsc.load_gather(idx_v, [_K * q16 + j])
            for i in range(_K):
                pos = (g_vec * (_K * _K) + (_K * j + i)) * _QG + qq
                plsc.store_scatter(rid_v, [pos], n_prompts * i + sel)

    rows = _CPW * _QG
    n_chunks = n_groups * _K * _K // _CPW
    ghandles = [None] * n_chunks
    whandles = [[] for _ in range(n_chunks)]
    expand(0)
    for c in range(min(_NBUF, n_chunks)):
        ghandles[c] = pltpu.async_copy(
            table_hbm.at[rid_v.at[pl.ds(c * rows, rows)]],
            bufs[c % _NBUF], gsems[c % _NBUF])
    for g16 in range(1, q_per_w // 16):
        expand(g16)
    for c in range(n_chunks):
        p = c % _NBUF
        ghandles[c].wait()
        for m in range(_CPW):
            g, co = divmod(c * _CPW + m, _K * _K)
            j, i = divmod(co, _K)
            whandles[c].append(pltpu.async_copy(
                bufs[p].at[pl.ds(m * _QG, _QG)],
                out_hbm.at[j, i, pl.ds(qb + g * _QG, _QG)], wsems[p]))
        nxt = c + _NBUF
        if nxt < n_chunks:
            for h in whandles[nxt - _NBUF]:
                h.wait()
            whandles[nxt - _NBUF] = []
            ghandles[nxt] = pltpu.async_copy(
                table_hbm.at[rid_v.at[pl.ds(nxt * rows, rows)]],
                bufs[nxt % _NBUF], gsems[nxt % _NBUF])
    for c in range(n_chunks):
        for h in whandles[c]:
            h.wait()


def kernel(query, top_k, prompts, prompt_keys):
    del top_k
    nq, d = query.shape
    n, k, _ = prompts.shape

    indices = pl.pallas_call(
        _simtopk_kernel,
        grid=(nq // _BQ,),
        in_specs=[
            pl.BlockSpec((_BQ, d), lambda i: (i, 0)),
            pl.BlockSpec((n, d), lambda i: (0, 0)),
        ],
        out_specs=pl.BlockSpec((_BQ, _K), lambda i: (i, 0)),
        out_shape=jax.ShapeDtypeStruct((nq, _K), jnp.int32),
    )(query, prompt_keys)

    b = nq * _K
    gather = functools.partial(
        pl.kernel,
        out_type=jax.ShapeDtypeStruct((_K, k, nq, d), jnp.float32),
        mesh=plsc.VectorSubcoreMesh(core_axis_name="c", subcore_axis_name="s"),
        compiler_params=pltpu.CompilerParams(
            needs_layout_passes=False, use_tc_tiling_on_sc=True),
        scratch_types=(
            [
                pltpu.VMEM((b // _NW,), jnp.int32),
                pltpu.VMEM((b // _NW * _K,), jnp.int32),
            ]
            + [pltpu.VMEM((_CPW * _QG, d), jnp.float32)] * _NBUF
            + [pltpu.SemaphoreType.DMA] * (2 * _NBUF)
        ),
    )(_sc_gather)
    table = jnp.transpose(prompts, (1, 0, 2)).reshape(k * n, d)
    gathered = gather(table, indices.reshape(b))

    return jnp.transpose(gathered, (2, 0, 1, 3)), indices
